# Initial kernel scaffold; baseline (speedup 1.0000x reference)
import jax
import jax.numpy as jnp
from jax.experimental import pallas as pl


def kernel(x_s, edge_index_s, edge_attr_s, x_t, edge_index_t, edge_attr_t, xs_batch, xt_batch, params):
    raise NotImplementedError("write your pallas kernel here")


# speedup vs baseline: 1.1386x; 1.1386x over previous
import functools

import jax
import jax.numpy as jnp
from jax.experimental import pallas as pl
from jax.experimental.pallas import tpu as pltpu

N = 10000
E = 160000
D = 128


def _gatv2_jnp(p, x, edge_index, edge_attr, heads, out_ch):
    n = x.shape[0]
    src = edge_index[0]
    dst = edge_index[1]
    xl = (x @ p["Wl"] + p["bl"]).reshape(n, heads, out_ch)
    xr = (x @ p["Wr"] + p["br"]).reshape(n, heads, out_ch)
    ee = (edge_attr @ p["We"]).reshape(-1, heads, out_ch)
    m = xl[dst] + xr[src] + ee
    m = jax.nn.leaky_relu(m, 0.2)
    logits = (m * p["att"][None, :, :]).sum(-1)
    w = jnp.exp(logits)
    den = jax.ops.segment_sum(w, dst, num_segments=n)
    out = jax.ops.segment_sum(xr[src] * w[..., None], dst, num_segments=n)
    out = out / (den[..., None] + 1e-16)
    return out.reshape(n, heads * out_ch) + p["bo"]


def _graph_norm_jnp(x, eps=1e-5):
    mean = jnp.mean(x, axis=0, keepdims=True)
    xc = x - mean
    var = jnp.mean(xc * xc, axis=0, keepdims=True)
    return xc / jnp.sqrt(var + eps)


def _pool_jnp(x, batch, num_graphs):
    s = jax.ops.segment_sum(x, batch, num_segments=num_graphs)
    cnt = jax.ops.segment_sum(jnp.ones((x.shape[0], 1), x.dtype), batch,
                              num_segments=num_graphs)
    return s / jnp.maximum(cnt, 1.0)


def _mlp_head_body(xs_ref, w1_ref, b1_ref, g1_ref, be1_ref, w2_ref, b2_ref,
                   out_ref, sig_ref):
    h = jnp.dot(xs_ref[...], w1_ref[...],
                preferred_element_type=jnp.float32) + b1_ref[...]
    mu = jnp.mean(h, axis=0, keepdims=True)
    hc = h - mu
    var = jnp.mean(hc * hc, axis=0, keepdims=True)
    h = hc / jnp.sqrt(var + 1e-5) * g1_ref[...] + be1_ref[...]
    out = jnp.dot(h, w2_ref[...],
                  preferred_element_type=jnp.float32) + b2_ref[...]
    out_ref[...] = out
    sig_ref[...] = jax.nn.sigmoid(out)


def _mlp_head(xs, params):
    out_shape = (
        jax.ShapeDtypeStruct((64, 1317), jnp.float32),
        jax.ShapeDtypeStruct((64, 1317), jnp.float32),
    )
    return pl.pallas_call(
        _mlp_head_body,
        out_shape=out_shape,
    )(xs, params["W1"], params["b1"].reshape(1, -1),
      params["gamma1"].reshape(1, -1), params["beta1"].reshape(1, -1),
      params["W2"], params["b2"].reshape(1, -1))


def kernel(x_s, edge_index_s, edge_attr_s, x_t, edge_index_t, edge_attr_t,
           xs_batch, xt_batch, params):
    x1 = _gatv2_jnp(params["s1"], x_s, edge_index_s, edge_attr_s, 8, D // 2)
    x1 = _graph_norm_jnp(x1)
    x2 = _gatv2_jnp(params["s2"], x1, edge_index_s, edge_attr_s, 1, D // 4)
    x2 = _graph_norm_jnp(x2)
    xs = _pool_jnp(x2, xs_batch, 64)
    return _mlp_head(xs, params)


# speedup vs baseline: 5.2577x; 4.6176x over previous
import functools

import jax
import jax.numpy as jnp
from jax import lax
from jax.experimental import pallas as pl
from jax.experimental.pallas import tpu as pltpu
from jax.experimental.pallas import tpu_sc as plsc

N = 10000
E = 160000
D = 128


def _proj2_body(x_ref, wl_ref, bl_ref, wr_ref, br_ref, xl_ref, xr_ref):
    x = x_ref[...]
    xl_ref[...] = jnp.dot(x, wl_ref[...],
                          preferred_element_type=jnp.float32) + bl_ref[...]
    xr_ref[...] = jnp.dot(x, wr_ref[...],
                          preferred_element_type=jnp.float32) + br_ref[...]


def _proj2(x, wl, bl, wr, br, blk=2000):
    n, din = x.shape
    dout = wl.shape[1]
    grid = (n // blk,)
    return pl.pallas_call(
        _proj2_body,
        grid=grid,
        in_specs=[
            pl.BlockSpec((blk, din), lambda i: (i, 0)),
            pl.BlockSpec((din, dout), lambda i: (0, 0)),
            pl.BlockSpec((1, dout), lambda i: (0, 0)),
            pl.BlockSpec((din, dout), lambda i: (0, 0)),
            pl.BlockSpec((1, dout), lambda i: (0, 0)),
        ],
        out_specs=[
            pl.BlockSpec((blk, dout), lambda i: (i, 0)),
            pl.BlockSpec((blk, dout), lambda i: (i, 0)),
        ],
        out_shape=[
            jax.ShapeDtypeStruct((n, dout), jnp.float32),
            jax.ShapeDtypeStruct((n, dout), jnp.float32),
        ],
    )(x, wl, bl.reshape(1, -1), wr, br.reshape(1, -1))


def _ee_body(ea_ref, we_ref, out_ref):
    out_ref[...] = jnp.dot(ea_ref[...], we_ref[...],
                           preferred_element_type=jnp.float32)


def _ee(edge_attr, we, blk=4000):
    e, k = edge_attr.shape
    dout = we.shape[1]
    return pl.pallas_call(
        _ee_body,
        grid=(e // blk,),
        in_specs=[
            pl.BlockSpec((blk, k), lambda i: (i, 0)),
            pl.BlockSpec((k, dout), lambda i: (0, 0)),
        ],
        out_specs=pl.BlockSpec((blk, dout), lambda i: (i, 0)),
        out_shape=jax.ShapeDtypeStruct((e, dout), jnp.float32),
    )(edge_attr, we)


def _fin1a_body(m0_ref, m1_ref, m2_ref, m3_ref, d0_ref, d1_ref, d2_ref,
                d3_ref, bo_ref, h_ref, s1_ref, s2_ref):
    parts = []
    for m_ref, d_ref in ((m0_ref, d0_ref), (m1_ref, d1_ref),
                         (m2_ref, d2_ref), (m3_ref, d3_ref)):
        msg = m_ref[0]
        d0 = jnp.broadcast_to(d_ref[0, :, 0:1], (1000, 64))
        d1 = jnp.broadcast_to(d_ref[0, :, 64:65], (1000, 64))
        den = jnp.concatenate([d0, d1], axis=1)
        parts.append(msg / (den + 1e-16))
    h = jnp.concatenate(parts, axis=1) + bo_ref[...]
    h_ref[...] = h
    s1 = jnp.sum(h, axis=0, keepdims=True)

    @pl.when(pl.program_id(0) == 0)
    def _init():
        s1_ref[...] = jnp.zeros_like(s1_ref)
        s2_ref[...] = jnp.zeros_like(s2_ref)

    s1_ref[0:1, :] += s1
    s2_ref[0:1, :] += jnp.sum(h * h, axis=0, keepdims=True)


def _fin1v_body(h_ref, s1_ref, v_ref):
    mu = s1_ref[0:1, :] * (1.0 / N)
    hc = h_ref[...] - mu

    @pl.when(pl.program_id(0) == 0)
    def _init():
        v_ref[...] = jnp.zeros_like(v_ref)

    v_ref[0:1, :] += jnp.sum(hc * hc, axis=0, keepdims=True)


def _fin1b_body(h_ref, s1_ref, v_ref, out_ref):
    mu = s1_ref[0:1, :] * (1.0 / N)
    var = v_ref[0:1, :] * (1.0 / N)
    out_ref[...] = (h_ref[...] - mu) / jnp.sqrt(var + 1e-5)


def _fin1(accg, bo, blk=1000):
    gspec = pl.BlockSpec((1, blk, 128), lambda i: (i // 5, i % 5, 0))
    h, s1, _ = pl.pallas_call(
        _fin1a_body,
        grid=(N // blk,),
        in_specs=[
            gspec, gspec, gspec, gspec, gspec, gspec, gspec, gspec,
            pl.BlockSpec((1, 512), lambda i: (0, 0)),
        ],
        out_specs=[
            pl.BlockSpec((blk, 512), lambda i: (i, 0)),
            pl.BlockSpec((8, 512), lambda i: (0, 0)),
            pl.BlockSpec((8, 512), lambda i: (0, 0)),
        ],
        out_shape=[
            jax.ShapeDtypeStruct((N, 512), jnp.float32),
            jax.ShapeDtypeStruct((8, 512), jnp.float32),
            jax.ShapeDtypeStruct((8, 512), jnp.float32),
        ],
    )(*accg, bo.reshape(1, -1))
    v = pl.pallas_call(
        _fin1v_body,
        grid=(N // blk,),
        in_specs=[
            pl.BlockSpec((blk, 512), lambda i: (i, 0)),
            pl.BlockSpec((8, 512), lambda i: (0, 0)),
        ],
        out_specs=pl.BlockSpec((8, 512), lambda i: (0, 0)),
        out_shape=jax.ShapeDtypeStruct((8, 512), jnp.float32),
    )(h, s1)
    return pl.pallas_call(
        _fin1b_body,
        grid=(N // blk,),
        in_specs=[
            pl.BlockSpec((blk, 512), lambda i: (i, 0)),
            pl.BlockSpec((8, 512), lambda i: (0, 0)),
            pl.BlockSpec((8, 512), lambda i: (0, 0)),
        ],
        out_specs=pl.BlockSpec((blk, 512), lambda i: (i, 0)),
        out_shape=jax.ShapeDtypeStruct((N, 512), jnp.float32),
    )(h, s1, v)


def _fin2_body(acc_ref, bo_ref, batch_ref, x2_ref, xs_ref):
    accf = acc_ref[0, :, :33] + acc_ref[1, :, :33]
    acc = accf[:, :32]
    den = accf[:, 32:33]
    h = acc / (den + 1e-16) + bo_ref[...]
    mu = jnp.mean(h, axis=0, keepdims=True)
    hc = h - mu
    var = jnp.mean(hc * hc, axis=0, keepdims=True)
    x2 = hc / jnp.sqrt(var + 1e-5)
    x2_ref[...] = x2
    oht = (lax.broadcasted_iota(jnp.int32, (64, N), 0)
           == batch_ref[...]).astype(jnp.float32)
    sums = jnp.dot(oht, x2, preferred_element_type=jnp.float32, precision=lax.Precision.HIGHEST)
    cnt = jnp.sum(oht, axis=1, keepdims=True)
    xs_ref[...] = sums / jnp.maximum(cnt, 1.0)


def _fin2pool(acc2x48, bo, batch):
    return pl.pallas_call(
        _fin2_body,
        out_shape=[
            jax.ShapeDtypeStruct((N, 32), jnp.float32),
            jax.ShapeDtypeStruct((64, 32), jnp.float32),
        ],
    )(acc2x48, bo.reshape(1, -1), batch.reshape(1, -1))


def _mlp_head_body(xs_ref, w1_ref, b1_ref, g1_ref, be1_ref, w2_ref, b2_ref,
                   out_ref, sig_ref):
    h = jnp.dot(xs_ref[...], w1_ref[...],
                preferred_element_type=jnp.float32) + b1_ref[...]
    mu = jnp.mean(h, axis=0, keepdims=True)
    hc = h - mu
    var = jnp.mean(hc * hc, axis=0, keepdims=True)
    h = hc / jnp.sqrt(var + 1e-5) * g1_ref[...] + be1_ref[...]
    out = jnp.dot(h, w2_ref[...],
                  preferred_element_type=jnp.float32) + b2_ref[...]
    out_ref[...] = out
    sig_ref[...] = jax.nn.sigmoid(out)


def _mlp_head(xs, params):
    out_shape = (
        jax.ShapeDtypeStruct((64, 1317), jnp.float32),
        jax.ShapeDtypeStruct((64, 1317), jnp.float32),
    )
    return pl.pallas_call(
        _mlp_head_body,
        out_shape=out_shape,
    )(xs, params["W1"], params["b1"].reshape(1, -1),
      params["gamma1"].reshape(1, -1), params["beta1"].reshape(1, -1),
      params["W2"], params["b2"].reshape(1, -1))


_L2B = 40
_L2SH = E // 32


def _edge2_body(src_hbm, dst_hbm, xl_hbm, xr_hbm, ee_hbm, att_hbm, out_hbm,
                src_b, dst_b, att_v, lbuf, rbuf, ebuf, msgbuf, zbuf,
                acc, sem):
    c = lax.axis_index("c")
    s = lax.axis_index("s")
    w = c * 16 + s
    base = w * _L2SH

    def _z(i, _):
        for j in range(8):
            zbuf[i, pl.ds(16 * j, 16)] = jnp.zeros((16,), jnp.float32)
        return 0

    lax.fori_loop(0, 104, _z, 0)
    for r in range(6):
        pltpu.sync_copy(zbuf, acc.at[pl.ds(s * 624 + r * 104, 104)])

    @pl.when(s == 0)
    def _ztail():
        pltpu.sync_copy(zbuf.at[pl.ds(0, 16)], acc.at[pl.ds(9984, 16)])

    pltpu.sync_copy(att_hbm, att_v)

    def _zm(i, _):
        for j in range(8):
            msgbuf[i, pl.ds(16 * j, 16)] = jnp.zeros((16,), jnp.float32)
        return 0

    lax.fori_loop(0, _L2B, _zm, 0)
    plsc.subcore_barrier()

    def _batch(b, carry):
        eb = base + b * _L2B
        pltpu.sync_copy(src_hbm.at[pl.ds(eb, _L2B)], src_b.at[0])
        pltpu.sync_copy(dst_hbm.at[pl.ds(eb, _L2B)], dst_b.at[0])
        cp1 = pltpu.async_copy(xl_hbm.at[dst_b.at[0]], lbuf, sem)
        cp2 = pltpu.async_copy(xr_hbm.at[src_b.at[0]], rbuf, sem)
        cp3 = pltpu.async_copy(ee_hbm.at[pl.ds(eb, _L2B)], ebuf, sem)
        cp1.wait()
        cp2.wait()
        cp3.wait()

        def _edge(i, carry):
            a0 = att_v[pl.ds(0, 16)]
            a1 = att_v[pl.ds(16, 16)]
            z0 = lbuf[i, pl.ds(0, 16)] + rbuf[i, pl.ds(0, 16)] \
                + ebuf[i, pl.ds(0, 16)]
            z1 = lbuf[i, pl.ds(16, 16)] + rbuf[i, pl.ds(16, 16)] \
                + ebuf[i, pl.ds(16, 16)]
            m0 = jnp.where(z0 > 0, z0, 0.2 * z0)
            m1 = jnp.where(z1 > 0, z1, 0.2 * z1)
            logit = jnp.sum(m0 * a0 + m1 * a1)
            wv = jnp.exp(jnp.full((16,), logit, jnp.float32))
            msgbuf[i, pl.ds(0, 16)] = rbuf[i, pl.ds(0, 16)] * wv
            msgbuf[i, pl.ds(16, 16)] = rbuf[i, pl.ds(16, 16)] * wv
            msgbuf[i, pl.ds(32, 16)] = jnp.where(
                lax.iota(jnp.int32, 16) == 0, wv, 0.0)
            return 0

        lax.fori_loop(0, _L2B, _edge, 0)
        pltpu.sync_copy(msgbuf, acc.at[dst_b.at[0]], add=True)
        return 0

    lax.fori_loop(0, _L2SH // _L2B, _batch, 0)
    plsc.subcore_barrier()
    pltpu.sync_copy(acc.at[pl.ds(s * 624, 624)],
                    out_hbm.at[c, pl.ds(s * 624, 624)])

    @pl.when(s == 0)
    def _wtail():
        pltpu.sync_copy(acc.at[pl.ds(9984, 16)],
                        out_hbm.at[c, pl.ds(9984, 16)])


def _edge2_sc(src, dst, xl2, xr2, ee2, att):
    mesh = plsc.VectorSubcoreMesh(core_axis_name="c", subcore_axis_name="s")
    f = pl.kernel(
        _edge2_body,
        out_type=jax.ShapeDtypeStruct((2, N, 128), jnp.float32),
        mesh=mesh,
        compiler_params=pltpu.CompilerParams(needs_layout_passes=False),
        scratch_types=[
            pltpu.VMEM((1, _L2B), jnp.int32),
            pltpu.VMEM((1, _L2B), jnp.int32),
            pltpu.VMEM((32,), jnp.float32),
            pltpu.VMEM((_L2B, 128), jnp.float32),
            pltpu.VMEM((_L2B, 128), jnp.float32),
            pltpu.VMEM((_L2B, 128), jnp.float32),
            pltpu.VMEM((_L2B, 128), jnp.float32),
            pltpu.VMEM((104, 128), jnp.float32),
            pltpu.VMEM_SHARED((N, 128), jnp.float32),
            pltpu.SemaphoreType.DMA,
        ],
    )
    return f(src, dst, xl2, xr2, ee2, att)


def _sidx_body(d_ref, s0_ref, s1_ref):
    d = d_ref[...]
    s0_ref[...] = jnp.where(d < 5000, d, 5000)
    s1_ref[...] = jnp.where(d >= 5000, d - 5000, 5000)


def _sidx(dst):
    s0, s1 = pl.pallas_call(
        _sidx_body,
        grid=(25,),
        in_specs=[pl.BlockSpec((1, 6400), lambda i: (0, i))],
        out_specs=[pl.BlockSpec((1, 6400), lambda i: (0, i)),
                   pl.BlockSpec((1, 6400), lambda i: (0, i))],
        out_shape=[jax.ShapeDtypeStruct((1, E), jnp.int32),
                   jax.ShapeDtypeStruct((1, E), jnp.int32)],
    )(dst.reshape(1, E))
    return s0.reshape(E), s1.reshape(E)


_L1B = 40
_L1SH = E // 16


def _edge1_body(src_hbm, dst_hbm, si0_hbm, si1_hbm,
                xl0, xl1, xl2, xl3, xr0, xr1, xr2, xr3,
                ee0, ee1, ee2, ee3, att_hbm,
                om0, om1, om2, om3, od0, od1, od2, od3,
                src_b, dst_b, sidx, att_v, lbuf, rbuf, ebuf, msgA, msgD,
                accM, accD, sem):
    c = lax.axis_index("c")
    s = lax.axis_index("s")
    nlo = c * 5000
    tbase = s * _L1SH

    pltpu.sync_copy(att_hbm, att_v)
    xls = [xl0, xl1, xl2, xl3]
    xrs = [xr0, xr1, xr2, xr3]
    ees = [ee0, ee1, ee2, ee3]
    outms = [om0, om1, om2, om3]
    outds = [od0, od1, od2, od3]

    for g in range(4):
        def _zm(i, _):
            for j in range(8):
                msgA[i, pl.ds(16 * j, 16)] = jnp.zeros((16,), jnp.float32)
                msgD[i, pl.ds(16 * j, 16)] = jnp.zeros((16,), jnp.float32)
            return 0

        lax.fori_loop(0, _L1B, _zm, 0)
        for buf, acc in ((msgA, accM), (msgD, accD)):
            for r in range(7):
                pltpu.sync_copy(buf, acc.at[pl.ds(s * 312 + r * 40, 40)])
            pltpu.sync_copy(buf.at[pl.ds(0, 32)],
                            acc.at[pl.ds(s * 312 + 280, 32)])

            @pl.when(s == 0)
            def _ztail():
                pltpu.sync_copy(buf.at[pl.ds(0, 16)],
                                acc.at[pl.ds(4992, 16)])

        plsc.subcore_barrier()

        def _batch(b, _):
            eb = tbase + b * _L1B
            pltpu.sync_copy(src_hbm.at[pl.ds(eb, _L1B)], src_b.at[0])
            pltpu.sync_copy(dst_hbm.at[pl.ds(eb, _L1B)], dst_b.at[0])
            cp1 = pltpu.async_copy(xls[g].at[dst_b.at[0]], lbuf, sem)
            cp2 = pltpu.async_copy(xrs[g].at[src_b.at[0]], rbuf, sem)
            cp3 = pltpu.async_copy(ees[g].at[pl.ds(eb, _L1B)], ebuf, sem)

            @pl.when(c == 0)
            def _si0():
                pltpu.sync_copy(si0_hbm.at[pl.ds(eb, _L1B)], sidx.at[0])

            @pl.when(c == 1)
            def _si1():
                pltpu.sync_copy(si1_hbm.at[pl.ds(eb, _L1B)], sidx.at[0])

            cp1.wait()
            cp2.wait()
            cp3.wait()

            def _edge(i, _):
                s0 = jnp.zeros((16,), jnp.float32)
                s1 = jnp.zeros((16,), jnp.float32)
                for v in range(8):
                    z = lbuf[i, pl.ds(16 * v, 16)] \
                        + rbuf[i, pl.ds(16 * v, 16)] \
                        + ebuf[i, pl.ds(16 * v, 16)]
                    m = jnp.where(z > 0, z, 0.2 * z)
                    t = m * att_v[pl.ds(128 * g + 16 * v, 16)]
                    if v < 4:
                        s0 = s0 + t
                    else:
                        s1 = s1 + t
                w0 = jnp.exp(jnp.full((16,), jnp.sum(s0), jnp.float32))
                w1 = jnp.exp(jnp.full((16,), jnp.sum(s1), jnp.float32))
                for v in range(8):
                    wv = w0 if v < 4 else w1
                    msgA[i, pl.ds(16 * v, 16)] = \
                        rbuf[i, pl.ds(16 * v, 16)] * wv
                lane0 = lax.iota(jnp.int32, 16) == 0
                msgD[i, pl.ds(0, 16)] = jnp.where(lane0, w0, 0.0)
                msgD[i, pl.ds(64, 16)] = jnp.where(lane0, w1, 0.0)
                return 0

            lax.fori_loop(0, _L1B, _edge, 0)
            pltpu.sync_copy(msgA, accM.at[sidx.at[0]], add=True)
            pltpu.sync_copy(msgD, accD.at[sidx.at[0]], add=True)
            return 0

        lax.fori_loop(0, _L1SH // _L1B, _batch, 0)
        plsc.subcore_barrier()
        for acc, outg in ((accM, outms[g]), (accD, outds[g])):
            pltpu.sync_copy(acc.at[pl.ds(s * 312, 312)],
                            outg.at[c, pl.ds(s * 312, 312)])

            @pl.when(s == 0)
            def _wtail():
                pltpu.sync_copy(acc.at[pl.ds(4992, 8)],
                                outg.at[c, pl.ds(4992, 8)])

        plsc.subcore_barrier()


def _edge1_sc(src, dst, si0, si1, xlg, xrg, eeg, att):
    mesh = plsc.VectorSubcoreMesh(core_axis_name="c", subcore_axis_name="s")
    f = pl.kernel(
        _edge1_body,
        out_type=[jax.ShapeDtypeStruct((2, 5000, 128), jnp.float32)] * 8,
        mesh=mesh,
        compiler_params=pltpu.CompilerParams(needs_layout_passes=False),
        scratch_types=[
            pltpu.VMEM((1, _L1B), jnp.int32),
            pltpu.VMEM((1, _L1B), jnp.int32),
            pltpu.VMEM((1, _L1B), jnp.int32),
            pltpu.VMEM((512,), jnp.float32),
            pltpu.VMEM((_L1B, 128), jnp.float32),
            pltpu.VMEM((_L1B, 128), jnp.float32),
            pltpu.VMEM((_L1B, 128), jnp.float32),
            pltpu.VMEM((_L1B, 128), jnp.float32),
            pltpu.VMEM((_L1B, 128), jnp.float32),
            pltpu.VMEM_SHARED((5008, 128), jnp.float32),
            pltpu.VMEM_SHARED((5008, 128), jnp.float32),
            pltpu.SemaphoreType.DMA,
        ],
    )
    return f(src, dst, si0, si1, *xlg, *xrg, *eeg, att)


def _edge_pass_jnp(xl, xr, ee, src, dst, att, heads, out_ch):
    n = xl.shape[0]
    m = (xl.reshape(n, heads, out_ch)[dst]
         + xr.reshape(n, heads, out_ch)[src]
         + ee.reshape(-1, heads, out_ch))
    m = jax.nn.leaky_relu(m, 0.2)
    logits = (m * att[None, :, :]).sum(-1)
    w = jnp.exp(logits)
    den = jax.ops.segment_sum(w, dst, num_segments=n)
    out = jax.ops.segment_sum(
        xr.reshape(n, heads, out_ch)[src] * w[..., None], dst, num_segments=n)
    acc = jnp.concatenate(
        [out.reshape(n, heads * out_ch), den,
         jnp.zeros((n, 16 - heads), jnp.float32)], axis=1)
    return acc


def kernel(x_s, edge_index_s, edge_attr_s, x_t, edge_index_t, edge_attr_t,
           xs_batch, xt_batch, params):
    p1, p2 = params["s1"], params["s2"]
    src = edge_index_s[0]
    dst = edge_index_s[1]

    xl1g, xr1g, ee1g = [], [], []
    for g in range(4):
        sl = slice(128 * g, 128 * (g + 1))
        xlg, xrg = _proj2(x_s, p1["Wl"][:, sl], p1["bl"][sl],
                          p1["Wr"][:, sl], p1["br"][sl])
        xl1g.append(xlg)
        xr1g.append(xrg)
        ee1g.append(_ee(edge_attr_s, p1["We"][:, sl]))
    si0, si1 = _sidx(dst)
    acc1 = _edge1_sc(src, dst, si0, si1, xl1g, xr1g, ee1g,
                     p1["att"].reshape(-1))
    x1 = _fin1(acc1, p1["bo"])

    pad = ((0, 0), (0, 96))
    xl2, xr2 = _proj2(x1, jnp.pad(p2["Wl"], pad), jnp.pad(p2["bl"], (0, 96)),
                      jnp.pad(p2["Wr"], pad), jnp.pad(p2["br"], (0, 96)))
    ee2 = _ee(edge_attr_s, jnp.pad(p2["We"], pad))
    acc2 = _edge2_sc(src, dst, xl2, xr2, ee2, p2["att"].reshape(-1))
    _, xs = _fin2pool(acc2, p2["bo"], xs_batch)

    return _mlp_head(xs, params)


# speedup vs baseline: 5.8630x; 1.1151x over previous
import functools

import jax
import jax.numpy as jnp
from jax import lax
from jax.experimental import pallas as pl
from jax.experimental.pallas import tpu as pltpu
from jax.experimental.pallas import tpu_sc as plsc

N = 10000
E = 160000
D = 128


def _proj2_body(x_ref, wl_ref, bl_ref, wr_ref, br_ref, xl_ref, xr_ref):
    x = x_ref[...]
    xl_ref[...] = jnp.dot(x, wl_ref[...],
                          preferred_element_type=jnp.float32) + bl_ref[...]
    xr_ref[...] = jnp.dot(x, wr_ref[...],
                          preferred_element_type=jnp.float32) + br_ref[...]


def _proj2(x, wl, bl, wr, br, blk=2000):
    n, din = x.shape
    dout = wl.shape[1]
    grid = (n // blk,)
    return pl.pallas_call(
        _proj2_body,
        grid=grid,
        in_specs=[
            pl.BlockSpec((blk, din), lambda i: (i, 0)),
            pl.BlockSpec((din, dout), lambda i: (0, 0)),
            pl.BlockSpec((1, dout), lambda i: (0, 0)),
            pl.BlockSpec((din, dout), lambda i: (0, 0)),
            pl.BlockSpec((1, dout), lambda i: (0, 0)),
        ],
        out_specs=[
            pl.BlockSpec((blk, dout), lambda i: (i, 0)),
            pl.BlockSpec((blk, dout), lambda i: (i, 0)),
        ],
        out_shape=[
            jax.ShapeDtypeStruct((n, dout), jnp.float32),
            jax.ShapeDtypeStruct((n, dout), jnp.float32),
        ],
    )(x, wl, bl.reshape(1, -1), wr, br.reshape(1, -1))


def _ee_body(ea_ref, we_ref, out_ref):
    out_ref[...] = jnp.dot(ea_ref[...], we_ref[...],
                           preferred_element_type=jnp.float32)


def _ee(edge_attr, we, blk=4000):
    e, k = edge_attr.shape
    dout = we.shape[1]
    return pl.pallas_call(
        _ee_body,
        grid=(e // blk,),
        in_specs=[
            pl.BlockSpec((blk, k), lambda i: (i, 0)),
            pl.BlockSpec((k, dout), lambda i: (0, 0)),
        ],
        out_specs=pl.BlockSpec((blk, dout), lambda i: (i, 0)),
        out_shape=jax.ShapeDtypeStruct((e, dout), jnp.float32),
    )(edge_attr, we)


def _fin1a_body(m0_ref, m1_ref, m2_ref, m3_ref, d0_ref, d1_ref, d2_ref,
                d3_ref, bo_ref, h_ref, s1_ref, s2_ref):
    parts = []
    for m_ref, d_ref in ((m0_ref, d0_ref), (m1_ref, d1_ref),
                         (m2_ref, d2_ref), (m3_ref, d3_ref)):
        msg = m_ref[0]
        d0 = jnp.broadcast_to(d_ref[0, :, 0:1], (1000, 64))
        d1 = jnp.broadcast_to(d_ref[0, :, 64:65], (1000, 64))
        den = jnp.concatenate([d0, d1], axis=1)
        parts.append(msg / (den + 1e-16))
    h = jnp.concatenate(parts, axis=1) + bo_ref[...]
    h_ref[...] = h
    s1 = jnp.sum(h, axis=0, keepdims=True)

    @pl.when(pl.program_id(0) == 0)
    def _init():
        s1_ref[...] = jnp.zeros_like(s1_ref)
        s2_ref[...] = jnp.zeros_like(s2_ref)

    s1_ref[0:1, :] += s1
    s2_ref[0:1, :] += jnp.sum(h * h, axis=0, keepdims=True)


def _fin1v_body(h_ref, s1_ref, v_ref):
    mu = s1_ref[0:1, :] * (1.0 / N)
    hc = h_ref[...] - mu

    @pl.when(pl.program_id(0) == 0)
    def _init():
        v_ref[...] = jnp.zeros_like(v_ref)

    v_ref[0:1, :] += jnp.sum(hc * hc, axis=0, keepdims=True)


def _fin1b_body(h_ref, s1_ref, v_ref, out_ref):
    mu = s1_ref[0:1, :] * (1.0 / N)
    var = v_ref[0:1, :] * (1.0 / N)
    out_ref[...] = (h_ref[...] - mu) / jnp.sqrt(var + 1e-5)


def _fin1(accg, bo, blk=1000):
    gspec = pl.BlockSpec((1, blk, 128), lambda i: (i // 5, i % 5, 0))
    h, s1, _ = pl.pallas_call(
        _fin1a_body,
        grid=(N // blk,),
        in_specs=[
            gspec, gspec, gspec, gspec, gspec, gspec, gspec, gspec,
            pl.BlockSpec((1, 512), lambda i: (0, 0)),
        ],
        out_specs=[
            pl.BlockSpec((blk, 512), lambda i: (i, 0)),
            pl.BlockSpec((8, 512), lambda i: (0, 0)),
            pl.BlockSpec((8, 512), lambda i: (0, 0)),
        ],
        out_shape=[
            jax.ShapeDtypeStruct((N, 512), jnp.float32),
            jax.ShapeDtypeStruct((8, 512), jnp.float32),
            jax.ShapeDtypeStruct((8, 512), jnp.float32),
        ],
    )(*accg, bo.reshape(1, -1))
    v = pl.pallas_call(
        _fin1v_body,
        grid=(N // blk,),
        in_specs=[
            pl.BlockSpec((blk, 512), lambda i: (i, 0)),
            pl.BlockSpec((8, 512), lambda i: (0, 0)),
        ],
        out_specs=pl.BlockSpec((8, 512), lambda i: (0, 0)),
        out_shape=jax.ShapeDtypeStruct((8, 512), jnp.float32),
    )(h, s1)
    return pl.pallas_call(
        _fin1b_body,
        grid=(N // blk,),
        in_specs=[
            pl.BlockSpec((blk, 512), lambda i: (i, 0)),
            pl.BlockSpec((8, 512), lambda i: (0, 0)),
            pl.BlockSpec((8, 512), lambda i: (0, 0)),
        ],
        out_specs=pl.BlockSpec((blk, 512), lambda i: (i, 0)),
        out_shape=jax.ShapeDtypeStruct((N, 512), jnp.float32),
    )(h, s1, v)


def _fin2_body(acc_ref, bo_ref, batch_ref, x2_ref, xs_ref):
    accf = acc_ref[0, :, :33] + acc_ref[1, :, :33]
    acc = accf[:, :32]
    den = accf[:, 32:33]
    h = acc / (den + 1e-16) + bo_ref[...]
    mu = jnp.mean(h, axis=0, keepdims=True)
    hc = h - mu
    var = jnp.mean(hc * hc, axis=0, keepdims=True)
    x2 = hc / jnp.sqrt(var + 1e-5)
    x2_ref[...] = x2
    oht = (lax.broadcasted_iota(jnp.int32, (64, N), 0)
           == batch_ref[...]).astype(jnp.float32)
    sums = jnp.dot(oht, x2, preferred_element_type=jnp.float32, precision=lax.Precision.HIGHEST)
    cnt = jnp.sum(oht, axis=1, keepdims=True)
    xs_ref[...] = sums / jnp.maximum(cnt, 1.0)


def _fin2pool(acc2x48, bo, batch):
    return pl.pallas_call(
        _fin2_body,
        out_shape=[
            jax.ShapeDtypeStruct((N, 32), jnp.float32),
            jax.ShapeDtypeStruct((64, 32), jnp.float32),
        ],
    )(acc2x48, bo.reshape(1, -1), batch.reshape(1, -1))


def _mlp_head_body(xs_ref, w1_ref, b1_ref, g1_ref, be1_ref, w2_ref, b2_ref,
                   out_ref, sig_ref):
    h = jnp.dot(xs_ref[...], w1_ref[...],
                preferred_element_type=jnp.float32) + b1_ref[...]
    mu = jnp.mean(h, axis=0, keepdims=True)
    hc = h - mu
    var = jnp.mean(hc * hc, axis=0, keepdims=True)
    h = hc / jnp.sqrt(var + 1e-5) * g1_ref[...] + be1_ref[...]
    out = jnp.dot(h, w2_ref[...],
                  preferred_element_type=jnp.float32) + b2_ref[...]
    out_ref[...] = out
    sig_ref[...] = jax.nn.sigmoid(out)


def _mlp_head(xs, params):
    out_shape = (
        jax.ShapeDtypeStruct((64, 1317), jnp.float32),
        jax.ShapeDtypeStruct((64, 1317), jnp.float32),
    )
    return pl.pallas_call(
        _mlp_head_body,
        out_shape=out_shape,
    )(xs, params["W1"], params["b1"].reshape(1, -1),
      params["gamma1"].reshape(1, -1), params["beta1"].reshape(1, -1),
      params["W2"], params["b2"].reshape(1, -1))


_L2B = 40
_L2SH = E // 32


def _edge2_body(src_hbm, dst_hbm, xl_hbm, xr_hbm, ee_hbm, att_hbm, out_hbm,
                src_b, dst_b, att_v, lbuf, rbuf, ebuf, msgbuf, zbuf,
                acc, sem):
    c = lax.axis_index("c")
    s = lax.axis_index("s")
    w = c * 16 + s
    base = w * _L2SH

    def _z(i, _):
        for j in range(8):
            zbuf[i, pl.ds(16 * j, 16)] = jnp.zeros((16,), jnp.float32)
        return 0

    lax.fori_loop(0, 104, _z, 0)
    for r in range(6):
        pltpu.sync_copy(zbuf, acc.at[pl.ds(s * 624 + r * 104, 104)])

    @pl.when(s == 0)
    def _ztail():
        pltpu.sync_copy(zbuf.at[pl.ds(0, 16)], acc.at[pl.ds(9984, 16)])

    pltpu.sync_copy(att_hbm, att_v)

    def _zm(i, _):
        for j in range(8):
            msgbuf[i, pl.ds(16 * j, 16)] = jnp.zeros((16,), jnp.float32)
        return 0

    lax.fori_loop(0, _L2B, _zm, 0)
    plsc.subcore_barrier()

    def _batch(b, carry):
        eb = base + b * _L2B
        ci1 = pltpu.async_copy(src_hbm.at[pl.ds(eb, _L2B)], src_b.at[0],
                               sem)
        ci2 = pltpu.async_copy(dst_hbm.at[pl.ds(eb, _L2B)], dst_b.at[0],
                               sem)
        ci1.wait()
        ci2.wait()
        cp1 = pltpu.async_copy(xl_hbm.at[dst_b.at[0]], lbuf, sem)
        cp2 = pltpu.async_copy(xr_hbm.at[src_b.at[0]], rbuf, sem)
        cp3 = pltpu.async_copy(ee_hbm.at[pl.ds(eb, _L2B)], ebuf, sem)
        cp1.wait()
        cp2.wait()
        cp3.wait()

        def _edge(i, carry):
            a0 = att_v[pl.ds(0, 16)]
            a1 = att_v[pl.ds(16, 16)]
            z0 = lbuf[i, pl.ds(0, 16)] + rbuf[i, pl.ds(0, 16)] \
                + ebuf[i, pl.ds(0, 16)]
            z1 = lbuf[i, pl.ds(16, 16)] + rbuf[i, pl.ds(16, 16)] \
                + ebuf[i, pl.ds(16, 16)]
            m0 = jnp.where(z0 > 0, z0, 0.2 * z0)
            m1 = jnp.where(z1 > 0, z1, 0.2 * z1)
            logit = jnp.sum(m0 * a0 + m1 * a1)
            wv = jnp.exp(jnp.full((16,), logit, jnp.float32))
            msgbuf[i, pl.ds(0, 16)] = rbuf[i, pl.ds(0, 16)] * wv
            msgbuf[i, pl.ds(16, 16)] = rbuf[i, pl.ds(16, 16)] * wv
            msgbuf[i, pl.ds(32, 16)] = jnp.where(
                lax.iota(jnp.int32, 16) == 0, wv, 0.0)
            return 0

        lax.fori_loop(0, _L2B, _edge, 0)
        pltpu.sync_copy(msgbuf, acc.at[dst_b.at[0]], add=True)
        return 0

    lax.fori_loop(0, _L2SH // _L2B, _batch, 0)
    plsc.subcore_barrier()
    pltpu.sync_copy(acc.at[pl.ds(s * 624, 624)],
                    out_hbm.at[c, pl.ds(s * 624, 624)])

    @pl.when(s == 0)
    def _wtail():
        pltpu.sync_copy(acc.at[pl.ds(9984, 16)],
                        out_hbm.at[c, pl.ds(9984, 16)])


def _edge2_sc(src, dst, xl2, xr2, ee2, att):
    mesh = plsc.VectorSubcoreMesh(core_axis_name="c", subcore_axis_name="s")
    f = pl.kernel(
        _edge2_body,
        out_type=jax.ShapeDtypeStruct((2, N, 128), jnp.float32),
        mesh=mesh,
        compiler_params=pltpu.CompilerParams(needs_layout_passes=False),
        scratch_types=[
            pltpu.VMEM((1, _L2B), jnp.int32),
            pltpu.VMEM((1, _L2B), jnp.int32),
            pltpu.VMEM((32,), jnp.float32),
            pltpu.VMEM((_L2B, 128), jnp.float32),
            pltpu.VMEM((_L2B, 128), jnp.float32),
            pltpu.VMEM((_L2B, 128), jnp.float32),
            pltpu.VMEM((_L2B, 128), jnp.float32),
            pltpu.VMEM((104, 128), jnp.float32),
            pltpu.VMEM_SHARED((N, 128), jnp.float32),
            pltpu.SemaphoreType.DMA,
        ],
    )
    return f(src, dst, xl2, xr2, ee2, att)


def _sidx_body(d_ref, s0_ref, s1_ref):
    d = d_ref[...]
    s0_ref[...] = jnp.where(d < 5000, d, 5000)
    s1_ref[...] = jnp.where(d >= 5000, d - 5000, 5000)


def _sidx(dst):
    s0, s1 = pl.pallas_call(
        _sidx_body,
        grid=(25,),
        in_specs=[pl.BlockSpec((1, 6400), lambda i: (0, i))],
        out_specs=[pl.BlockSpec((1, 6400), lambda i: (0, i)),
                   pl.BlockSpec((1, 6400), lambda i: (0, i))],
        out_shape=[jax.ShapeDtypeStruct((1, E), jnp.int32),
                   jax.ShapeDtypeStruct((1, E), jnp.int32)],
    )(dst.reshape(1, E))
    return s0.reshape(E), s1.reshape(E)


_L1B = 40
_L1SH = E // 16


def _edge1_body(src_hbm, dst_hbm, si0_hbm, si1_hbm,
                xl0, xl1, xl2, xl3, xr0, xr1, xr2, xr3,
                ee0, ee1, ee2, ee3, att_hbm,
                om0, om1, om2, om3, od0, od1, od2, od3,
                src_b, dst_b, sidx, att_v, lbuf, rbuf, ebuf, msgA, msgD,
                accM, accD, sem):
    c = lax.axis_index("c")
    s = lax.axis_index("s")
    nlo = c * 5000
    tbase = s * _L1SH

    pltpu.sync_copy(att_hbm, att_v)
    xls = [xl0, xl1, xl2, xl3]
    xrs = [xr0, xr1, xr2, xr3]
    ees = [ee0, ee1, ee2, ee3]
    outms = [om0, om1, om2, om3]
    outds = [od0, od1, od2, od3]

    for g in range(4):
        def _zm(i, _):
            for j in range(8):
                msgA[i, pl.ds(16 * j, 16)] = jnp.zeros((16,), jnp.float32)
                msgD[i, pl.ds(16 * j, 16)] = jnp.zeros((16,), jnp.float32)
            return 0

        lax.fori_loop(0, _L1B, _zm, 0)
        for buf, acc in ((msgA, accM), (msgD, accD)):
            for r in range(7):
                pltpu.sync_copy(buf, acc.at[pl.ds(s * 312 + r * 40, 40)])
            pltpu.sync_copy(buf.at[pl.ds(0, 32)],
                            acc.at[pl.ds(s * 312 + 280, 32)])

            @pl.when(s == 0)
            def _ztail():
                pltpu.sync_copy(buf.at[pl.ds(0, 16)],
                                acc.at[pl.ds(4992, 16)])

        plsc.subcore_barrier()

        def _batch(b, _):
            eb = tbase + b * _L1B
            ci1 = pltpu.async_copy(src_hbm.at[pl.ds(eb, _L1B)],
                                   src_b.at[0], sem)
            ci2 = pltpu.async_copy(dst_hbm.at[pl.ds(eb, _L1B)],
                                   dst_b.at[0], sem)

            @pl.when(c == 0)
            def _si0():
                pltpu.async_copy(si0_hbm.at[pl.ds(eb, _L1B)], sidx.at[0],
                                 sem).wait()

            @pl.when(c == 1)
            def _si1():
                pltpu.async_copy(si1_hbm.at[pl.ds(eb, _L1B)], sidx.at[0],
                                 sem).wait()

            ci1.wait()
            ci2.wait()
            cp1 = pltpu.async_copy(xls[g].at[dst_b.at[0]], lbuf, sem)
            cp2 = pltpu.async_copy(xrs[g].at[src_b.at[0]], rbuf, sem)
            cp3 = pltpu.async_copy(ees[g].at[pl.ds(eb, _L1B)], ebuf, sem)
            cp1.wait()
            cp2.wait()
            cp3.wait()

            def _edge(i, _):
                s0 = jnp.zeros((16,), jnp.float32)
                s1 = jnp.zeros((16,), jnp.float32)
                for v in range(8):
                    z = lbuf[i, pl.ds(16 * v, 16)] \
                        + rbuf[i, pl.ds(16 * v, 16)] \
                        + ebuf[i, pl.ds(16 * v, 16)]
                    m = jnp.where(z > 0, z, 0.2 * z)
                    t = m * att_v[pl.ds(128 * g + 16 * v, 16)]
                    if v < 4:
                        s0 = s0 + t
                    else:
                        s1 = s1 + t
                w0 = jnp.exp(jnp.full((16,), jnp.sum(s0), jnp.float32))
                w1 = jnp.exp(jnp.full((16,), jnp.sum(s1), jnp.float32))
                for v in range(8):
                    wv = w0 if v < 4 else w1
                    msgA[i, pl.ds(16 * v, 16)] = \
                        rbuf[i, pl.ds(16 * v, 16)] * wv
                lane0 = lax.iota(jnp.int32, 16) == 0
                msgD[i, pl.ds(0, 16)] = jnp.where(lane0, w0, 0.0)
                msgD[i, pl.ds(64, 16)] = jnp.where(lane0, w1, 0.0)
                return 0

            lax.fori_loop(0, _L1B, _edge, 0)
            cs1 = pltpu.async_copy(msgA, accM.at[sidx.at[0]], sem,
                                   add=True)
            cs2 = pltpu.async_copy(msgD, accD.at[sidx.at[0]], sem,
                                   add=True)
            cs1.wait()
            cs2.wait()
            return 0

        lax.fori_loop(0, _L1SH // _L1B, _batch, 0)
        plsc.subcore_barrier()
        for acc, outg in ((accM, outms[g]), (accD, outds[g])):
            pltpu.sync_copy(acc.at[pl.ds(s * 312, 312)],
                            outg.at[c, pl.ds(s * 312, 312)])

            @pl.when(s == 0)
            def _wtail():
                pltpu.sync_copy(acc.at[pl.ds(4992, 8)],
                                outg.at[c, pl.ds(4992, 8)])

        plsc.subcore_barrier()


def _edge1_sc(src, dst, si0, si1, xlg, xrg, eeg, att):
    mesh = plsc.VectorSubcoreMesh(core_axis_name="c", subcore_axis_name="s")
    f = pl.kernel(
        _edge1_body,
        out_type=[jax.ShapeDtypeStruct((2, 5000, 128), jnp.float32)] * 8,
        mesh=mesh,
        compiler_params=pltpu.CompilerParams(needs_layout_passes=False),
        scratch_types=[
            pltpu.VMEM((1, _L1B), jnp.int32),
            pltpu.VMEM((1, _L1B), jnp.int32),
            pltpu.VMEM((1, _L1B), jnp.int32),
            pltpu.VMEM((512,), jnp.float32),
            pltpu.VMEM((_L1B, 128), jnp.float32),
            pltpu.VMEM((_L1B, 128), jnp.float32),
            pltpu.VMEM((_L1B, 128), jnp.float32),
            pltpu.VMEM((_L1B, 128), jnp.float32),
            pltpu.VMEM((_L1B, 128), jnp.float32),
            pltpu.VMEM_SHARED((5008, 128), jnp.float32),
            pltpu.VMEM_SHARED((5008, 128), jnp.float32),
            pltpu.SemaphoreType.DMA,
        ],
    )
    return f(src, dst, si0, si1, *xlg, *xrg, *eeg, att)


def _edge_pass_jnp(xl, xr, ee, src, dst, att, heads, out_ch):
    n = xl.shape[0]
    m = (xl.reshape(n, heads, out_ch)[dst]
         + xr.reshape(n, heads, out_ch)[src]
         + ee.reshape(-1, heads, out_ch))
    m = jax.nn.leaky_relu(m, 0.2)
    logits = (m * att[None, :, :]).sum(-1)
    w = jnp.exp(logits)
    den = jax.ops.segment_sum(w, dst, num_segments=n)
    out = jax.ops.segment_sum(
        xr.reshape(n, heads, out_ch)[src] * w[..., None], dst, num_segments=n)
    acc = jnp.concatenate(
        [out.reshape(n, heads * out_ch), den,
         jnp.zeros((n, 16 - heads), jnp.float32)], axis=1)
    return acc


def kernel(x_s, edge_index_s, edge_attr_s, x_t, edge_index_t, edge_attr_t,
           xs_batch, xt_batch, params):
    p1, p2 = params["s1"], params["s2"]
    src = edge_index_s[0]
    dst = edge_index_s[1]

    xl1g, xr1g, ee1g = [], [], []
    for g in range(4):
        sl = slice(128 * g, 128 * (g + 1))
        xlg, xrg = _proj2(x_s, p1["Wl"][:, sl], p1["bl"][sl],
                          p1["Wr"][:, sl], p1["br"][sl])
        xl1g.append(xlg)
        xr1g.append(xrg)
        ee1g.append(_ee(edge_attr_s, p1["We"][:, sl]))
    si0, si1 = _sidx(dst)
    acc1 = _edge1_sc(src, dst, si0, si1, xl1g, xr1g, ee1g,
                     p1["att"].reshape(-1))
    x1 = _fin1(acc1, p1["bo"])

    pad = ((0, 0), (0, 96))
    xl2, xr2 = _proj2(x1, jnp.pad(p2["Wl"], pad), jnp.pad(p2["bl"], (0, 96)),
                      jnp.pad(p2["Wr"], pad), jnp.pad(p2["br"], (0, 96)))
    ee2 = _ee(edge_attr_s, jnp.pad(p2["We"], pad))
    acc2 = _edge2_sc(src, dst, xl2, xr2, ee2, p2["att"].reshape(-1))
    _, xs = _fin2pool(acc2, p2["bo"], xs_batch)

    return _mlp_head(xs, params)


# speedup vs baseline: 5.9274x; 1.0110x over previous
import functools

import jax
import jax.numpy as jnp
from jax import lax
from jax.experimental import pallas as pl
from jax.experimental.pallas import tpu as pltpu
from jax.experimental.pallas import tpu_sc as plsc

N = 10000
E = 160000
D = 128


def _proj2_body(x_ref, wl_ref, bl_ref, wr_ref, br_ref, xl_ref, xr_ref):
    x = x_ref[...]
    xl_ref[...] = jnp.dot(x, wl_ref[...],
                          preferred_element_type=jnp.float32) + bl_ref[...]
    xr_ref[...] = jnp.dot(x, wr_ref[...],
                          preferred_element_type=jnp.float32) + br_ref[...]


def _proj2(x, wl, bl, wr, br, blk=2000):
    n, din = x.shape
    dout = wl.shape[1]
    grid = (n // blk,)
    return pl.pallas_call(
        _proj2_body,
        grid=grid,
        in_specs=[
            pl.BlockSpec((blk, din), lambda i: (i, 0)),
            pl.BlockSpec((din, dout), lambda i: (0, 0)),
            pl.BlockSpec((1, dout), lambda i: (0, 0)),
            pl.BlockSpec((din, dout), lambda i: (0, 0)),
            pl.BlockSpec((1, dout), lambda i: (0, 0)),
        ],
        out_specs=[
            pl.BlockSpec((blk, dout), lambda i: (i, 0)),
            pl.BlockSpec((blk, dout), lambda i: (i, 0)),
        ],
        out_shape=[
            jax.ShapeDtypeStruct((n, dout), jnp.float32),
            jax.ShapeDtypeStruct((n, dout), jnp.float32),
        ],
    )(x, wl, bl.reshape(1, -1), wr, br.reshape(1, -1))


def _ee_body(ea_ref, we_ref, out_ref):
    out_ref[...] = jnp.dot(ea_ref[...], we_ref[...],
                           preferred_element_type=jnp.float32)


def _ee(edge_attr, we, blk=4000):
    e, k = edge_attr.shape
    dout = we.shape[1]
    return pl.pallas_call(
        _ee_body,
        grid=(e // blk,),
        in_specs=[
            pl.BlockSpec((blk, k), lambda i: (i, 0)),
            pl.BlockSpec((k, dout), lambda i: (0, 0)),
        ],
        out_specs=pl.BlockSpec((blk, dout), lambda i: (i, 0)),
        out_shape=jax.ShapeDtypeStruct((e, dout), jnp.float32),
    )(edge_attr, we)


def _fin1a_body(m0_ref, m1_ref, m2_ref, m3_ref, d0_ref, d1_ref, d2_ref,
                d3_ref, bo_ref, h_ref, s1_ref, s2_ref):
    parts = []
    for m_ref, d_ref in ((m0_ref, d0_ref), (m1_ref, d1_ref),
                         (m2_ref, d2_ref), (m3_ref, d3_ref)):
        msg = m_ref[0]
        d0 = jnp.broadcast_to(d_ref[0, :, 0:1], (1000, 64))
        d1 = jnp.broadcast_to(d_ref[0, :, 64:65], (1000, 64))
        den = jnp.concatenate([d0, d1], axis=1)
        parts.append(msg / (den + 1e-16))
    h = jnp.concatenate(parts, axis=1) + bo_ref[...]
    h_ref[...] = h
    s1 = jnp.sum(h, axis=0, keepdims=True)

    @pl.when(pl.program_id(0) == 0)
    def _init():
        s1_ref[...] = jnp.zeros_like(s1_ref)
        s2_ref[...] = jnp.zeros_like(s2_ref)

    s1_ref[0:1, :] += s1
    s2_ref[0:1, :] += jnp.sum(h * h, axis=0, keepdims=True)


def _fin1v_body(h_ref, s1_ref, v_ref):
    mu = s1_ref[0:1, :] * (1.0 / N)
    hc = h_ref[...] - mu

    @pl.when(pl.program_id(0) == 0)
    def _init():
        v_ref[...] = jnp.zeros_like(v_ref)

    v_ref[0:1, :] += jnp.sum(hc * hc, axis=0, keepdims=True)


def _fin1b_body(h_ref, s1_ref, v_ref, out_ref):
    mu = s1_ref[0:1, :] * (1.0 / N)
    var = v_ref[0:1, :] * (1.0 / N)
    out_ref[...] = (h_ref[...] - mu) / jnp.sqrt(var + 1e-5)


def _fin1(accg, bo, blk=1000):
    gspec = pl.BlockSpec((1, blk, 128), lambda i: (i // 5, i % 5, 0))
    h, s1, _ = pl.pallas_call(
        _fin1a_body,
        grid=(N // blk,),
        in_specs=[
            gspec, gspec, gspec, gspec, gspec, gspec, gspec, gspec,
            pl.BlockSpec((1, 512), lambda i: (0, 0)),
        ],
        out_specs=[
            pl.BlockSpec((blk, 512), lambda i: (i, 0)),
            pl.BlockSpec((8, 512), lambda i: (0, 0)),
            pl.BlockSpec((8, 512), lambda i: (0, 0)),
        ],
        out_shape=[
            jax.ShapeDtypeStruct((N, 512), jnp.float32),
            jax.ShapeDtypeStruct((8, 512), jnp.float32),
            jax.ShapeDtypeStruct((8, 512), jnp.float32),
        ],
    )(*accg, bo.reshape(1, -1))
    v = pl.pallas_call(
        _fin1v_body,
        grid=(N // blk,),
        in_specs=[
            pl.BlockSpec((blk, 512), lambda i: (i, 0)),
            pl.BlockSpec((8, 512), lambda i: (0, 0)),
        ],
        out_specs=pl.BlockSpec((8, 512), lambda i: (0, 0)),
        out_shape=jax.ShapeDtypeStruct((8, 512), jnp.float32),
    )(h, s1)
    return pl.pallas_call(
        _fin1b_body,
        grid=(N // blk,),
        in_specs=[
            pl.BlockSpec((blk, 512), lambda i: (i, 0)),
            pl.BlockSpec((8, 512), lambda i: (0, 0)),
            pl.BlockSpec((8, 512), lambda i: (0, 0)),
        ],
        out_specs=pl.BlockSpec((blk, 512), lambda i: (i, 0)),
        out_shape=jax.ShapeDtypeStruct((N, 512), jnp.float32),
    )(h, s1, v)


def _fin2_body(acc_ref, bo_ref, batch_ref, x2_ref, xs_ref):
    accf = acc_ref[0, :, :33] + acc_ref[1, :, :33]
    acc = accf[:, :32]
    den = accf[:, 32:33]
    h = acc / (den + 1e-16) + bo_ref[...]
    mu = jnp.mean(h, axis=0, keepdims=True)
    hc = h - mu
    var = jnp.mean(hc * hc, axis=0, keepdims=True)
    x2 = hc / jnp.sqrt(var + 1e-5)
    x2_ref[...] = x2
    oht = (lax.broadcasted_iota(jnp.int32, (64, N), 0)
           == batch_ref[...]).astype(jnp.float32)
    sums = jnp.dot(oht, x2, preferred_element_type=jnp.float32, precision=lax.Precision.HIGHEST)
    cnt = jnp.sum(oht, axis=1, keepdims=True)
    xs_ref[...] = sums / jnp.maximum(cnt, 1.0)


def _fin2pool(acc2x48, bo, batch):
    return pl.pallas_call(
        _fin2_body,
        out_shape=[
            jax.ShapeDtypeStruct((N, 32), jnp.float32),
            jax.ShapeDtypeStruct((64, 32), jnp.float32),
        ],
    )(acc2x48, bo.reshape(1, -1), batch.reshape(1, -1))


def _mlp_head_body(xs_ref, w1_ref, b1_ref, g1_ref, be1_ref, w2_ref, b2_ref,
                   out_ref, sig_ref):
    h = jnp.dot(xs_ref[...], w1_ref[...],
                preferred_element_type=jnp.float32) + b1_ref[...]
    mu = jnp.mean(h, axis=0, keepdims=True)
    hc = h - mu
    var = jnp.mean(hc * hc, axis=0, keepdims=True)
    h = hc / jnp.sqrt(var + 1e-5) * g1_ref[...] + be1_ref[...]
    out = jnp.dot(h, w2_ref[...],
                  preferred_element_type=jnp.float32) + b2_ref[...]
    out_ref[...] = out
    sig_ref[...] = jax.nn.sigmoid(out)


def _mlp_head(xs, params):
    out_shape = (
        jax.ShapeDtypeStruct((64, 1317), jnp.float32),
        jax.ShapeDtypeStruct((64, 1317), jnp.float32),
    )
    return pl.pallas_call(
        _mlp_head_body,
        out_shape=out_shape,
    )(xs, params["W1"], params["b1"].reshape(1, -1),
      params["gamma1"].reshape(1, -1), params["beta1"].reshape(1, -1),
      params["W2"], params["b2"].reshape(1, -1))


_L2B = 40
_L2SH = E // 32


def _edge2_body(src_hbm, dst_hbm, xl_hbm, xr_hbm, ee_hbm, att_hbm, out_hbm,
                src_b, dst_b, att_v, lbuf, rbuf, ebuf, msgbuf, zbuf,
                acc, sem):
    c = lax.axis_index("c")
    s = lax.axis_index("s")
    w = c * 16 + s
    base = w * _L2SH

    def _z(i, _):
        for j in range(8):
            zbuf[i, pl.ds(16 * j, 16)] = jnp.zeros((16,), jnp.float32)
        return 0

    lax.fori_loop(0, 104, _z, 0)
    for r in range(6):
        pltpu.sync_copy(zbuf, acc.at[pl.ds(s * 624 + r * 104, 104)])

    @pl.when(s == 0)
    def _ztail():
        pltpu.sync_copy(zbuf.at[pl.ds(0, 16)], acc.at[pl.ds(9984, 16)])

    pltpu.sync_copy(att_hbm, att_v)

    def _zm(i, _):
        for j in range(8):
            msgbuf[i, pl.ds(16 * j, 16)] = jnp.zeros((16,), jnp.float32)
        return 0

    lax.fori_loop(0, _L2B, _zm, 0)
    plsc.subcore_barrier()

    def _batch(b, carry):
        eb = base + b * _L2B
        ci1 = pltpu.async_copy(src_hbm.at[pl.ds(eb, _L2B)], src_b.at[0],
                               sem)
        ci2 = pltpu.async_copy(dst_hbm.at[pl.ds(eb, _L2B)], dst_b.at[0],
                               sem)
        cp3 = pltpu.async_copy(ee_hbm.at[pl.ds(eb, _L2B)], ebuf, sem)
        ci1.wait()
        ci2.wait()
        cp1 = pltpu.async_copy(xl_hbm.at[dst_b.at[0]], lbuf, sem)
        cp2 = pltpu.async_copy(xr_hbm.at[src_b.at[0]], rbuf, sem)
        cp1.wait()
        cp2.wait()
        cp3.wait()

        def _edge(i, carry):
            a0 = att_v[pl.ds(0, 16)]
            a1 = att_v[pl.ds(16, 16)]
            z0 = lbuf[i, pl.ds(0, 16)] + rbuf[i, pl.ds(0, 16)] \
                + ebuf[i, pl.ds(0, 16)]
            z1 = lbuf[i, pl.ds(16, 16)] + rbuf[i, pl.ds(16, 16)] \
                + ebuf[i, pl.ds(16, 16)]
            m0 = jnp.where(z0 > 0, z0, 0.2 * z0)
            m1 = jnp.where(z1 > 0, z1, 0.2 * z1)
            logit = jnp.sum(m0 * a0 + m1 * a1)
            wv = jnp.exp(jnp.full((16,), logit, jnp.float32))
            msgbuf[i, pl.ds(0, 16)] = rbuf[i, pl.ds(0, 16)] * wv
            msgbuf[i, pl.ds(16, 16)] = rbuf[i, pl.ds(16, 16)] * wv
            msgbuf[i, pl.ds(32, 16)] = jnp.where(
                lax.iota(jnp.int32, 16) == 0, wv, 0.0)
            return 0

        lax.fori_loop(0, _L2B, _edge, 0)
        pltpu.sync_copy(msgbuf, acc.at[dst_b.at[0]], add=True)
        return 0

    lax.fori_loop(0, _L2SH // _L2B, _batch, 0)
    plsc.subcore_barrier()
    pltpu.sync_copy(acc.at[pl.ds(s * 624, 624)],
                    out_hbm.at[c, pl.ds(s * 624, 624)])

    @pl.when(s == 0)
    def _wtail():
        pltpu.sync_copy(acc.at[pl.ds(9984, 16)],
                        out_hbm.at[c, pl.ds(9984, 16)])


def _edge2_sc(src, dst, xl2, xr2, ee2, att):
    mesh = plsc.VectorSubcoreMesh(core_axis_name="c", subcore_axis_name="s")
    f = pl.kernel(
        _edge2_body,
        out_type=jax.ShapeDtypeStruct((2, N, 128), jnp.float32),
        mesh=mesh,
        compiler_params=pltpu.CompilerParams(needs_layout_passes=False),
        scratch_types=[
            pltpu.VMEM((1, _L2B), jnp.int32),
            pltpu.VMEM((1, _L2B), jnp.int32),
            pltpu.VMEM((32,), jnp.float32),
            pltpu.VMEM((_L2B, 128), jnp.float32),
            pltpu.VMEM((_L2B, 128), jnp.float32),
            pltpu.VMEM((_L2B, 128), jnp.float32),
            pltpu.VMEM((_L2B, 128), jnp.float32),
            pltpu.VMEM((104, 128), jnp.float32),
            pltpu.VMEM_SHARED((N, 128), jnp.float32),
            pltpu.SemaphoreType.DMA,
        ],
    )
    return f(src, dst, xl2, xr2, ee2, att)


def _sidx_body(d_ref, s0_ref, s1_ref):
    d = d_ref[...]
    s0_ref[...] = jnp.where(d < 5000, d, 5000)
    s1_ref[...] = jnp.where(d >= 5000, d - 5000, 5000)


def _sidx(dst):
    s0, s1 = pl.pallas_call(
        _sidx_body,
        grid=(25,),
        in_specs=[pl.BlockSpec((1, 6400), lambda i: (0, i))],
        out_specs=[pl.BlockSpec((1, 6400), lambda i: (0, i)),
                   pl.BlockSpec((1, 6400), lambda i: (0, i))],
        out_shape=[jax.ShapeDtypeStruct((1, E), jnp.int32),
                   jax.ShapeDtypeStruct((1, E), jnp.int32)],
    )(dst.reshape(1, E))
    return s0.reshape(E), s1.reshape(E)


_L1B = 40
_L1SH = E // 16


def _edge1_body(src_hbm, dst_hbm, si0_hbm, si1_hbm,
                xl0, xl1, xl2, xl3, xr0, xr1, xr2, xr3,
                ee0, ee1, ee2, ee3, att_hbm,
                om0, om1, om2, om3, od0, od1, od2, od3,
                src_b, dst_b, sidx, att_v, lbuf, rbuf, ebuf, msgA, msgD,
                accM, accD, sem):
    c = lax.axis_index("c")
    s = lax.axis_index("s")
    nlo = c * 5000
    tbase = s * _L1SH

    pltpu.sync_copy(att_hbm, att_v)
    xls = [xl0, xl1, xl2, xl3]
    xrs = [xr0, xr1, xr2, xr3]
    ees = [ee0, ee1, ee2, ee3]
    outms = [om0, om1, om2, om3]
    outds = [od0, od1, od2, od3]

    for g in range(4):
        def _zm(i, _):
            for j in range(8):
                msgA[i, pl.ds(16 * j, 16)] = jnp.zeros((16,), jnp.float32)
                msgD[i, pl.ds(16 * j, 16)] = jnp.zeros((16,), jnp.float32)
            return 0

        lax.fori_loop(0, _L1B, _zm, 0)
        for buf, acc in ((msgA, accM), (msgD, accD)):
            for r in range(7):
                pltpu.sync_copy(buf, acc.at[pl.ds(s * 312 + r * 40, 40)])
            pltpu.sync_copy(buf.at[pl.ds(0, 32)],
                            acc.at[pl.ds(s * 312 + 280, 32)])

            @pl.when(s == 0)
            def _ztail():
                pltpu.sync_copy(buf.at[pl.ds(0, 16)],
                                acc.at[pl.ds(4992, 16)])

        plsc.subcore_barrier()

        def _batch(b, _):
            eb = tbase + b * _L1B
            ci1 = pltpu.async_copy(src_hbm.at[pl.ds(eb, _L1B)],
                                   src_b.at[0], sem)
            ci2 = pltpu.async_copy(dst_hbm.at[pl.ds(eb, _L1B)],
                                   dst_b.at[0], sem)

            @pl.when(c == 0)
            def _si0():
                pltpu.async_copy(si0_hbm.at[pl.ds(eb, _L1B)], sidx.at[0],
                                 sem).wait()

            @pl.when(c == 1)
            def _si1():
                pltpu.async_copy(si1_hbm.at[pl.ds(eb, _L1B)], sidx.at[0],
                                 sem).wait()

            cp3 = pltpu.async_copy(ees[g].at[pl.ds(eb, _L1B)], ebuf, sem)
            ci1.wait()
            ci2.wait()
            cp1 = pltpu.async_copy(xls[g].at[dst_b.at[0]], lbuf, sem)
            cp2 = pltpu.async_copy(xrs[g].at[src_b.at[0]], rbuf, sem)
            cp1.wait()
            cp2.wait()
            cp3.wait()

            def _edge(i, _):
                s0 = jnp.zeros((16,), jnp.float32)
                s1 = jnp.zeros((16,), jnp.float32)
                for v in range(8):
                    z = lbuf[i, pl.ds(16 * v, 16)] \
                        + rbuf[i, pl.ds(16 * v, 16)] \
                        + ebuf[i, pl.ds(16 * v, 16)]
                    m = jnp.where(z > 0, z, 0.2 * z)
                    t = m * att_v[pl.ds(128 * g + 16 * v, 16)]
                    if v < 4:
                        s0 = s0 + t
                    else:
                        s1 = s1 + t
                w0 = jnp.exp(jnp.full((16,), jnp.sum(s0), jnp.float32))
                w1 = jnp.exp(jnp.full((16,), jnp.sum(s1), jnp.float32))
                for v in range(8):
                    wv = w0 if v < 4 else w1
                    msgA[i, pl.ds(16 * v, 16)] = \
                        rbuf[i, pl.ds(16 * v, 16)] * wv
                lane0 = lax.iota(jnp.int32, 16) == 0
                msgD[i, pl.ds(0, 16)] = jnp.where(lane0, w0, 0.0)
                msgD[i, pl.ds(64, 16)] = jnp.where(lane0, w1, 0.0)
                return 0

            lax.fori_loop(0, _L1B, _edge, 0)
            cs1 = pltpu.async_copy(msgA, accM.at[sidx.at[0]], sem,
                                   add=True)
            cs2 = pltpu.async_copy(msgD, accD.at[sidx.at[0]], sem,
                                   add=True)
            cs1.wait()
            cs2.wait()
            return 0

        lax.fori_loop(0, _L1SH // _L1B, _batch, 0)
        plsc.subcore_barrier()
        for acc, outg in ((accM, outms[g]), (accD, outds[g])):
            pltpu.sync_copy(acc.at[pl.ds(s * 312, 312)],
                            outg.at[c, pl.ds(s * 312, 312)])

            @pl.when(s == 0)
            def _wtail():
                pltpu.sync_copy(acc.at[pl.ds(4992, 8)],
                                outg.at[c, pl.ds(4992, 8)])

        plsc.subcore_barrier()


def _edge1_sc(src, dst, si0, si1, xlg, xrg, eeg, att):
    mesh = plsc.VectorSubcoreMesh(core_axis_name="c", subcore_axis_name="s")
    f = pl.kernel(
        _edge1_body,
        out_type=[jax.ShapeDtypeStruct((2, 5000, 128), jnp.float32)] * 8,
        mesh=mesh,
        compiler_params=pltpu.CompilerParams(needs_layout_passes=False),
        scratch_types=[
            pltpu.VMEM((1, _L1B), jnp.int32),
            pltpu.VMEM((1, _L1B), jnp.int32),
            pltpu.VMEM((1, _L1B), jnp.int32),
            pltpu.VMEM((512,), jnp.float32),
            pltpu.VMEM((_L1B, 128), jnp.float32),
            pltpu.VMEM((_L1B, 128), jnp.float32),
            pltpu.VMEM((_L1B, 128), jnp.float32),
            pltpu.VMEM((_L1B, 128), jnp.float32),
            pltpu.VMEM((_L1B, 128), jnp.float32),
            pltpu.VMEM_SHARED((5008, 128), jnp.float32),
            pltpu.VMEM_SHARED((5008, 128), jnp.float32),
            pltpu.SemaphoreType.DMA,
        ],
    )
    return f(src, dst, si0, si1, *xlg, *xrg, *eeg, att)


def _edge_pass_jnp(xl, xr, ee, src, dst, att, heads, out_ch):
    n = xl.shape[0]
    m = (xl.reshape(n, heads, out_ch)[dst]
         + xr.reshape(n, heads, out_ch)[src]
         + ee.reshape(-1, heads, out_ch))
    m = jax.nn.leaky_relu(m, 0.2)
    logits = (m * att[None, :, :]).sum(-1)
    w = jnp.exp(logits)
    den = jax.ops.segment_sum(w, dst, num_segments=n)
    out = jax.ops.segment_sum(
        xr.reshape(n, heads, out_ch)[src] * w[..., None], dst, num_segments=n)
    acc = jnp.concatenate(
        [out.reshape(n, heads * out_ch), den,
         jnp.zeros((n, 16 - heads), jnp.float32)], axis=1)
    return acc


def kernel(x_s, edge_index_s, edge_attr_s, x_t, edge_index_t, edge_attr_t,
           xs_batch, xt_batch, params):
    p1, p2 = params["s1"], params["s2"]
    src = edge_index_s[0]
    dst = edge_index_s[1]

    xl1g, xr1g, ee1g = [], [], []
    for g in range(4):
        sl = slice(128 * g, 128 * (g + 1))
        xlg, xrg = _proj2(x_s, p1["Wl"][:, sl], p1["bl"][sl],
                          p1["Wr"][:, sl], p1["br"][sl])
        xl1g.append(xlg)
        xr1g.append(xrg)
        ee1g.append(_ee(edge_attr_s, p1["We"][:, sl]))
    si0, si1 = _sidx(dst)
    acc1 = _edge1_sc(src, dst, si0, si1, xl1g, xr1g, ee1g,
                     p1["att"].reshape(-1))
    x1 = _fin1(acc1, p1["bo"])

    pad = ((0, 0), (0, 96))
    xl2, xr2 = _proj2(x1, jnp.pad(p2["Wl"], pad), jnp.pad(p2["bl"], (0, 96)),
                      jnp.pad(p2["Wr"], pad), jnp.pad(p2["br"], (0, 96)))
    ee2 = _ee(edge_attr_s, jnp.pad(p2["We"], pad))
    acc2 = _edge2_sc(src, dst, xl2, xr2, ee2, p2["att"].reshape(-1))
    _, xs = _fin2pool(acc2, p2["bo"], xs_batch)

    return _mlp_head(xs, params)
